# Initial kernel scaffold; baseline (speedup 1.0000x reference)
import jax
import jax.numpy as jnp
from jax.experimental import pallas as pl


def kernel(x, edge_index, W_gcn, gamma_bn, beta_bn, W_film, b_film):
    raise NotImplementedError("write your pallas kernel here")


# speedup vs baseline: 8.4973x; 8.4973x over previous
import functools

import jax
import jax.numpy as jnp
from jax import lax
from jax.experimental import pallas as pl
from jax.experimental.pallas import tpu as pltpu
from jax.experimental.pallas import tpu_sc as plsc

N = 10000
E = 320000
D = 128
DH = D // 2
NC = 2
NS = 16
NW = NC * NS
EPT = E // NW
C = 80
NCH = EPT // C
NP = 10240
HP = 20480
RPT = NP // NS
RC = 128
NRC = RPT // RC
RH = HP // NS
BN_EPS = 1e-4

_mesh = plsc.VectorSubcoreMesh(core_axis_name="c", subcore_axis_name="s")
_sc_params = pltpu.CompilerParams(use_tc_tiling_on_sc=False)


@functools.partial(
    pl.kernel,
    out_type=jax.ShapeDtypeStruct((NC, HP, 16), jnp.float32),
    mesh=_mesh,
    compiler_params=_sc_params,
    scratch_types=[
        pltpu.VMEM((C,), jnp.int32),
        pltpu.VMEM((C,), jnp.int32),
        pltpu.VMEM((C, 16), jnp.float32),
        pltpu.VMEM((RH, 16), jnp.float32),
        pltpu.VMEM_SHARED((HP, 16), jnp.float32),
    ],
)
def _sc_degrees(src_hbm, dst_hbm, deg_hbm, idx_v, idx2_v, ones_v, buf_v, deg_sh):
    c = lax.axis_index("c")
    s = lax.axis_index("s")
    base = (c * NS + s) * EPT

    def fill(i, carry):
        buf_v[i, :] = jnp.zeros((16,), jnp.float32)
        return carry

    lax.fori_loop(0, RH, fill, 0)

    def fill_ones(i, carry):
        ones_v[i, :] = jnp.ones((16,), jnp.float32)
        return carry

    lax.fori_loop(0, C, fill_ones, 0)
    pltpu.sync_copy(buf_v, deg_sh.at[pl.ds(s * RH, RH)])
    plsc.subcore_barrier()

    def chunk(ch, carry):
        e0 = pl.multiple_of(base + ch * C, 8)
        pltpu.sync_copy(dst_hbm.at[pl.ds(e0, C)], idx_v)
        pltpu.sync_copy(ones_v, deg_sh.at[idx_v], add=True)
        pltpu.sync_copy(src_hbm.at[pl.ds(e0, C)], idx_v)
        for j in range(C // 16):
            idx2_v[pl.ds(j * 16, 16)] = idx_v[pl.ds(j * 16, 16)] + N
        pltpu.sync_copy(ones_v, deg_sh.at[idx2_v], add=True)
        return carry

    lax.fori_loop(0, NCH, chunk, 0)
    plsc.subcore_barrier()
    pltpu.sync_copy(deg_sh.at[pl.ds(s * RH, RH)], buf_v)
    pltpu.sync_copy(buf_v, deg_hbm.at[c, pl.ds(s * RH, RH)])


@functools.partial(
    pl.kernel,
    out_type=jax.ShapeDtypeStruct((2, NC, NP, DH), jnp.float32),
    mesh=_mesh,
    compiler_params=_sc_params,
    scratch_types=[
        pltpu.VMEM((C,), jnp.int32),
        pltpu.VMEM((C,), jnp.int32),
        pltpu.VMEM((C, DH), jnp.float32),
        pltpu.VMEM((RC, DH), jnp.float32),
        pltpu.VMEM_SHARED((NP, DH), jnp.float32),
    ],
)
def _sc_aggregate(xs_a, xs_b, src_hbm, dst_hbm, agg_hbm, src_v, dst_v, rows_v,
                  buf_v, agg_sh):
    c = lax.axis_index("c")
    s = lax.axis_index("s")
    base = (c * NS + s) * EPT

    def zb(i, carry):
        for j in range(DH // 16):
            buf_v[i, pl.ds(j * 16, 16)] = jnp.zeros((16,), jnp.float32)
        return carry

    for half, xs_hbm in enumerate((xs_a, xs_b)):
        lax.fori_loop(0, RC, zb, 0)
        for k in range(NRC):
            pltpu.sync_copy(buf_v, agg_sh.at[pl.ds(s * RPT + k * RC, RC)])
        plsc.subcore_barrier()

        def chunk(ch, carry):
            e0 = pl.multiple_of(base + ch * C, 8)
            pltpu.sync_copy(src_hbm.at[pl.ds(e0, C)], src_v)
            pltpu.sync_copy(dst_hbm.at[pl.ds(e0, C)], dst_v)
            pltpu.sync_copy(xs_hbm.at[src_v], rows_v)
            pltpu.sync_copy(rows_v, agg_sh.at[dst_v], add=True)
            return carry

        lax.fori_loop(0, NCH, chunk, 0)
        plsc.subcore_barrier()
        for k in range(NRC):
            r0 = s * RPT + k * RC
            pltpu.sync_copy(agg_sh.at[pl.ds(r0, RC)], buf_v)
            pltpu.sync_copy(buf_v, agg_hbm.at[half, c, pl.ds(r0, RC)])
        plsc.subcore_barrier()


def _tc_prescale_body(x_ref, deg_ref, xs_a_ref, xs_b_ref):
    deg_out = deg_ref[0, N:2 * N, 0:1] + deg_ref[1, N:2 * N, 0:1]
    r_out = lax.rsqrt(jnp.maximum(deg_out, 1.0))
    xs = x_ref[...] * r_out
    xs_a_ref[...] = xs[:, :DH]
    xs_b_ref[...] = xs[:, DH:]


RB = 1000
NB = N // RB


def _tc_gcn_body(agg_ref, deg_ref, wg_ref, y_ref, sums_ref):
    i = pl.program_id(0)
    deg_in = deg_ref[0, :, 0:1] + deg_ref[1, :, 0:1]
    r_in = lax.rsqrt(jnp.maximum(deg_in, 1.0))
    agg_lo = agg_ref[0, 0] + agg_ref[0, 1]
    agg_hi = agg_ref[1, 0] + agg_ref[1, 1]
    agg = jnp.concatenate([agg_lo, agg_hi], axis=1) * r_in
    y = jnp.dot(agg, wg_ref[...], preferred_element_type=jnp.float32,
                precision=lax.Precision.HIGHEST)
    y_ref[...] = y

    @pl.when(i == 0)
    def _():
        sums_ref[...] = jnp.zeros_like(sums_ref)

    sums_ref[0:1, :] += jnp.sum(y, axis=0, keepdims=True)
    sums_ref[1:2, :] += jnp.sum(y * y, axis=0, keepdims=True)


def _tc_film_body(y_ref, sums_ref, x_ref, gb_ref, bb_ref, wf_ref, bf_ref,
                  o_ref):
    y = y_ref[...]
    mean = sums_ref[0:1, :] * (1.0 / N)
    var = sums_ref[1:2, :] * (1.0 / N) - mean * mean
    yn = gb_ref[...] * (y - mean) * lax.rsqrt(var + BN_EPS) + bb_ref[...]
    film = jnp.dot(yn, wf_ref[...], preferred_element_type=jnp.float32,
                   precision=lax.Precision.HIGHEST) + bf_ref[...]
    z = film[:, :D] * yn + film[:, D:]
    o_ref[...] = jnp.maximum(z, 0.0) + x_ref[...]


def kernel(x, edge_index, W_gcn, gamma_bn, beta_bn, W_film, b_film):
    src = edge_index[0].astype(jnp.int32)
    dst = edge_index[1].astype(jnp.int32)
    deg = _sc_degrees(src, dst)
    xs_a, xs_b = pl.pallas_call(
        _tc_prescale_body,
        out_shape=(jax.ShapeDtypeStruct((N, DH), jnp.float32),
                   jax.ShapeDtypeStruct((N, DH), jnp.float32)),
    )(x, deg)
    agg = _sc_aggregate(xs_a, xs_b, src, dst)
    y, sums = pl.pallas_call(
        _tc_gcn_body,
        grid=(NB,),
        in_specs=[
            pl.BlockSpec((2, NC, RB, DH), lambda i: (0, 0, i, 0)),
            pl.BlockSpec((NC, RB, 16), lambda i: (0, i, 0)),
            pl.BlockSpec((D, D), lambda i: (0, 0)),
        ],
        out_specs=[
            pl.BlockSpec((RB, D), lambda i: (i, 0)),
            pl.BlockSpec((8, D), lambda i: (0, 0)),
        ],
        out_shape=(jax.ShapeDtypeStruct((N, D), jnp.float32),
                   jax.ShapeDtypeStruct((8, D), jnp.float32)),
    )(agg, deg, W_gcn)
    out = pl.pallas_call(
        _tc_film_body,
        grid=(NB,),
        in_specs=[
            pl.BlockSpec((RB, D), lambda i: (i, 0)),
            pl.BlockSpec((8, D), lambda i: (0, 0)),
            pl.BlockSpec((RB, D), lambda i: (i, 0)),
            pl.BlockSpec((1, D), lambda i: (0, 0)),
            pl.BlockSpec((1, D), lambda i: (0, 0)),
            pl.BlockSpec((D, 2 * D), lambda i: (0, 0)),
            pl.BlockSpec((1, 2 * D), lambda i: (0, 0)),
        ],
        out_specs=pl.BlockSpec((RB, D), lambda i: (i, 0)),
        out_shape=jax.ShapeDtypeStruct((N, D), jnp.float32),
    )(y, sums, x, gamma_bn.reshape(1, D), beta_bn.reshape(1, D), W_film,
      b_film.reshape(1, 2 * D))
    return out


# speedup vs baseline: 22.3976x; 2.6358x over previous
import functools

import jax
import jax.numpy as jnp
from jax import lax
from jax.experimental import pallas as pl
from jax.experimental.pallas import tpu as pltpu
from jax.experimental.pallas import tpu_sc as plsc

N = 10000
E = 320000
D = 128
DH = D // 2
NC = 2
NS = 16
NW = NC * NS
EPT = E // NW
C = 100
NCH = EPT // C
GRP = 10
NP = 10240
RPT = NP // NS
RC = 128
NRC = RPT // RC
BN_EPS = 1e-4

_mesh = plsc.VectorSubcoreMesh(core_axis_name="c", subcore_axis_name="s")
_sc_params = pltpu.CompilerParams(use_tc_tiling_on_sc=False)


@functools.partial(
    pl.kernel,
    out_type=jax.ShapeDtypeStruct((NC, 2, NP, 16), jnp.float32),
    mesh=_mesh,
    compiler_params=_sc_params,
    scratch_types=[
        pltpu.VMEM((NCH, C), jnp.int32),
        pltpu.VMEM((C, 16), jnp.float32),
        pltpu.VMEM((RPT, 16), jnp.float32),
        pltpu.VMEM_SHARED((NP, 16), jnp.float32),
        pltpu.VMEM_SHARED((NP, 16), jnp.float32),
        pltpu.SemaphoreType.DMA,
    ],
)
def _sc_degrees(src_hbm, dst_hbm, deg_hbm, idx_v, ones_v, buf_v, degin_sh,
                degout_sh, sem):
    c = lax.axis_index("c")
    s = lax.axis_index("s")
    wid = c * NS + s

    def fill(i, carry):
        buf_v[i, :] = jnp.zeros((16,), jnp.float32)
        return carry

    lax.fori_loop(0, RPT, fill, 0)

    def fill_ones(i, carry):
        ones_v[i, :] = jnp.ones((16,), jnp.float32)
        return carry

    lax.fori_loop(0, C, fill_ones, 0)
    pltpu.sync_copy(buf_v, degin_sh.at[pl.ds(s * RPT, RPT)])
    pltpu.sync_copy(buf_v, degout_sh.at[pl.ds(s * RPT, RPT)])
    plsc.subcore_barrier()

    for idx_hbm, deg_sh in ((dst_hbm, degin_sh), (src_hbm, degout_sh)):
        pltpu.sync_copy(idx_hbm.at[wid], idx_v)

        def group(g, carry):
            for k in range(GRP):
                pltpu.async_copy(ones_v, deg_sh.at[idx_v.at[g * GRP + k]],
                                 sem, add=True)
            for k in range(GRP):
                pltpu.make_async_copy(ones_v, deg_sh.at[idx_v.at[0]],
                                      sem).wait()
            return carry

        lax.fori_loop(0, NCH // GRP, group, 0)

    plsc.subcore_barrier()
    pltpu.sync_copy(degin_sh.at[pl.ds(s * RPT, RPT)], buf_v)
    pltpu.sync_copy(buf_v, deg_hbm.at[c, 0, pl.ds(s * RPT, RPT)])
    pltpu.sync_copy(degout_sh.at[pl.ds(s * RPT, RPT)], buf_v)
    pltpu.sync_copy(buf_v, deg_hbm.at[c, 1, pl.ds(s * RPT, RPT)])


@functools.partial(
    pl.kernel,
    out_type=jax.ShapeDtypeStruct((2, NC, NP, DH), jnp.float32),
    mesh=_mesh,
    compiler_params=_sc_params,
    scratch_types=[
        pltpu.VMEM((NCH, C), jnp.int32),
        pltpu.VMEM((NCH, C), jnp.int32),
        pltpu.VMEM((C, DH), jnp.float32),
        pltpu.VMEM((C, DH), jnp.float32),
        pltpu.VMEM((RC, DH), jnp.float32),
        pltpu.VMEM_SHARED((NP, DH), jnp.float32),
        pltpu.SemaphoreType.DMA,
    ],
)
def _sc_aggregate(xs_a, xs_b, src_hbm, dst_hbm, agg_hbm, srcs_v, dsts_v,
                  rows0_v, rows1_v, buf_v, agg_sh, sem):
    c = lax.axis_index("c")
    s = lax.axis_index("s")
    wid = c * NS + s
    pltpu.sync_copy(src_hbm.at[wid], srcs_v)
    pltpu.sync_copy(dst_hbm.at[wid], dsts_v)

    def zb(i, carry):
        for j in range(DH // 16):
            buf_v[i, pl.ds(j * 16, 16)] = jnp.zeros((16,), jnp.float32)
        return carry

    for half, xs_hbm in ((0, xs_a), (1, xs_b)):
        lax.fori_loop(0, RC, zb, 0)
        for k in range(NRC):
            pltpu.sync_copy(buf_v, agg_sh.at[pl.ds(s * RPT + k * RC, RC)])
        plsc.subcore_barrier()

        pltpu.async_copy(xs_hbm.at[srcs_v.at[0]], rows0_v, sem)

        def pair(h, carry):
            ch0 = 2 * h
            ch1 = ch0 + 1
            pltpu.async_copy(xs_hbm.at[srcs_v.at[ch1]], rows1_v, sem)
            pltpu.make_async_copy(xs_hbm.at[srcs_v.at[ch0]], rows0_v,
                                  sem).wait()
            pltpu.sync_copy(rows0_v, agg_sh.at[dsts_v.at[ch0]], add=True)

            @pl.when(ch1 + 1 < NCH)
            def _():
                pltpu.async_copy(xs_hbm.at[srcs_v.at[ch1 + 1]], rows0_v, sem)

            pltpu.make_async_copy(xs_hbm.at[srcs_v.at[ch1]], rows1_v,
                                  sem).wait()
            pltpu.sync_copy(rows1_v, agg_sh.at[dsts_v.at[ch1]], add=True)
            return carry

        lax.fori_loop(0, NCH // 2, pair, 0)
        plsc.subcore_barrier()
        for k in range(NRC):
            r0 = s * RPT + k * RC
            pltpu.sync_copy(agg_sh.at[pl.ds(r0, RC)], buf_v)
            pltpu.sync_copy(buf_v, agg_hbm.at[half, c, pl.ds(r0, RC)])
        plsc.subcore_barrier()


def _tc_prescale_body(x_ref, deg_ref, xs_a_ref, xs_b_ref):
    deg_out = deg_ref[0, 1, :N, 0:1] + deg_ref[1, 1, :N, 0:1]
    r_out = lax.rsqrt(jnp.maximum(deg_out, 1.0))
    xs = x_ref[...] * r_out
    xs_a_ref[...] = xs[:, :DH]
    xs_b_ref[...] = xs[:, DH:]


RB = 1000
NB = N // RB


def _tc_gcn_body(agg_ref, deg_ref, wg_ref, y_ref, sums_ref):
    i = pl.program_id(0)
    deg_in = deg_ref[0, 0, :, 0:1] + deg_ref[1, 0, :, 0:1]
    r_in = lax.rsqrt(jnp.maximum(deg_in, 1.0))
    agg_lo = agg_ref[0, 0] + agg_ref[0, 1]
    agg_hi = agg_ref[1, 0] + agg_ref[1, 1]
    agg = jnp.concatenate([agg_lo, agg_hi], axis=1) * r_in
    y = jnp.dot(agg, wg_ref[...], preferred_element_type=jnp.float32,
                precision=lax.Precision.HIGHEST)
    y_ref[...] = y

    @pl.when(i == 0)
    def _():
        sums_ref[...] = jnp.zeros_like(sums_ref)

    sums_ref[0:1, :] += jnp.sum(y, axis=0, keepdims=True)
    sums_ref[1:2, :] += jnp.sum(y * y, axis=0, keepdims=True)


def _tc_film_body(y_ref, sums_ref, x_ref, gb_ref, bb_ref, wf_ref, bf_ref,
                  o_ref):
    y = y_ref[...]
    mean = sums_ref[0:1, :] * (1.0 / N)
    var = sums_ref[1:2, :] * (1.0 / N) - mean * mean
    yn = gb_ref[...] * (y - mean) * lax.rsqrt(var + BN_EPS) + bb_ref[...]
    film = jnp.dot(yn, wf_ref[...], preferred_element_type=jnp.float32,
                   precision=lax.Precision.HIGHEST) + bf_ref[...]
    z = film[:, :D] * yn + film[:, D:]
    o_ref[...] = jnp.maximum(z, 0.0) + x_ref[...]


def kernel(x, edge_index, W_gcn, gamma_bn, beta_bn, W_film, b_film):
    src = edge_index[0].astype(jnp.int32).reshape(NW, NCH, C)
    dst = edge_index[1].astype(jnp.int32).reshape(NW, NCH, C)
    deg = _sc_degrees(src, dst)
    xs_a, xs_b = pl.pallas_call(
        _tc_prescale_body,
        out_shape=(jax.ShapeDtypeStruct((N, DH), jnp.float32),
                   jax.ShapeDtypeStruct((N, DH), jnp.float32)),
    )(x, deg)
    agg = _sc_aggregate(xs_a, xs_b, src, dst)
    y, sums = pl.pallas_call(
        _tc_gcn_body,
        grid=(NB,),
        in_specs=[
            pl.BlockSpec((2, NC, RB, DH), lambda i: (0, 0, i, 0)),
            pl.BlockSpec((NC, 2, RB, 16), lambda i: (0, 0, i, 0)),
            pl.BlockSpec((D, D), lambda i: (0, 0)),
        ],
        out_specs=[
            pl.BlockSpec((RB, D), lambda i: (i, 0)),
            pl.BlockSpec((8, D), lambda i: (0, 0)),
        ],
        out_shape=(jax.ShapeDtypeStruct((N, D), jnp.float32),
                   jax.ShapeDtypeStruct((8, D), jnp.float32)),
    )(agg, deg, W_gcn)
    out = pl.pallas_call(
        _tc_film_body,
        grid=(NB,),
        in_specs=[
            pl.BlockSpec((RB, D), lambda i: (i, 0)),
            pl.BlockSpec((8, D), lambda i: (0, 0)),
            pl.BlockSpec((RB, D), lambda i: (i, 0)),
            pl.BlockSpec((1, D), lambda i: (0, 0)),
            pl.BlockSpec((1, D), lambda i: (0, 0)),
            pl.BlockSpec((D, 2 * D), lambda i: (0, 0)),
            pl.BlockSpec((1, 2 * D), lambda i: (0, 0)),
        ],
        out_specs=pl.BlockSpec((RB, D), lambda i: (i, 0)),
        out_shape=jax.ShapeDtypeStruct((N, D), jnp.float32),
    )(y, sums, x, gamma_bn.reshape(1, D), beta_bn.reshape(1, D), W_film,
      b_film.reshape(1, 2 * D))
    return out
